# baseline (speedup 1.0000x reference)
import functools

import jax
import jax.numpy as jnp
import numpy as np
from jax.experimental import pallas as pl
from jax.experimental.pallas import tpu as pltpu

SLOTS = 1024
MDIM = 64
MHEADS = 4
TOPK = 32
VOCAB = 8192
EDIM = 512
HDIM = 512
NHATTN = 8
DFF = 2048
B = 4
T = 32

C_RK = 0
C_WK = 256
C_WV = 384
C_ER = 512
C_AG = 640
NCOL = 768
KEY_OFFS = (C_RK, C_RK + 64, C_RK + 128, C_RK + 192, C_WK)


def _scan_body(h_ref, wl_ref, ws_ref, bias_ref, dec_ref, out_ref,
               mem_ref, plog_ref, psm_ref):
    h = h_ref[...]
    plog_ref[...] = (jax.lax.dot_general(
        h, wl_ref[:HDIM], (((1,), (0,)), ((), ())),
        preferred_element_type=jnp.float32) + bias_ref[...]).reshape(T, B, VOCAB)
    psm_ref[...] = jax.lax.dot_general(
        h, ws_ref[:HDIM], (((1,), (0,)), ((), ())),
        preferred_element_type=jnp.float32).reshape(T, B, NCOL)
    mem_ref[...] = jnp.zeros((B, MDIM, SLOTS), jnp.float32)
    dec = dec_ref[...]
    wl2 = wl_ref[HDIM:]
    ws2 = ws_ref[HDIM:]

    lane_iota = jax.lax.broadcasted_iota(jnp.int32, (5 * B, 128), 1)

    def step(t, rv):
        out_ref[t] = plog_ref[t] + jax.lax.dot_general(
            rv, wl2, (((1,), (0,)), ((), ())), preferred_element_type=jnp.float32)
        pr = psm_ref[t] + jax.lax.dot_general(
            rv, ws2, (((1,), (0,)), ((), ())),
            preferred_element_type=jnp.float32)

        s_rows = []
        for b in range(B):
            kb = jnp.concatenate(
                [pr[b:b + 1, o:o + MDIM] for o in KEY_OFFS], axis=0)
            s_rows.append(jax.lax.dot_general(
                kb, mem_ref[b], (((1,), (0,)), ((), ())),
                preferred_element_type=jnp.float32))
        s_orig = jnp.concatenate(s_rows, axis=0)

        def fold_max(chunks):
            a = jnp.maximum(jnp.maximum(chunks[0], chunks[1]),
                            jnp.maximum(chunks[2], chunks[3]))
            b = jnp.maximum(jnp.maximum(chunks[4], chunks[5]),
                            jnp.maximum(chunks[6], chunks[7]))
            return jnp.maximum(a, b)

        s_ch = [s_orig[:, 128 * j:128 * j + 128] for j in range(8)]
        sel_ch = [jnp.zeros((5 * B, 128), jnp.float32) for _ in range(8)]
        r = fold_max(s_ch)
        for _ in range(TOPK):
            m = jnp.max(r, axis=1, keepdims=True)
            t_lane = jnp.min(jnp.where(r == m, lane_iota, jnp.int32(2 ** 30)),
                             axis=1, keepdims=True)
            ohl = lane_iota == t_lane
            seen = jnp.zeros((5 * B, 128), jnp.bool_)
            for j in range(8):
                cand = jnp.logical_and(s_ch[j] == m, ohl)
                take = jnp.logical_and(cand, jnp.logical_not(seen))
                seen = jnp.logical_or(seen, cand)
                s_ch[j] = jnp.where(take, jnp.float32(-1e30), s_ch[j])
                sel_ch[j] = jnp.where(take, 1.0, sel_ch[j])
            r = fold_max(s_ch)
        sel = jnp.concatenate(sel_ch, axis=1)

        gmax = jnp.max(s_orig, axis=1, keepdims=True)
        w_un = sel * jnp.exp(s_orig - gmax)
        wf = w_un / jnp.sum(w_un, axis=1, keepdims=True)

        wv_t = jnp.transpose(pr[:, C_WV:C_WV + MDIM])
        er_t = jnp.transpose(jax.nn.sigmoid(pr[:, C_ER:C_ER + MDIM]))
        ag = jax.nn.sigmoid(pr[:, C_AG:C_AG + 1])

        rv_rows = []
        for b in range(B):
            mb = mem_ref[b]
            wr = wf[5 * b:5 * b + MHEADS]
            rb = jax.lax.dot_general(
                wr, mb, (((1,), (1,)), ((), ())),
                preferred_element_type=jnp.float32)
            rv_rows.append(jnp.mean(rb, axis=0, keepdims=True))
            ww = wf[5 * b + MHEADS:5 * b + MHEADS + 1]
            upd = mb * (1.0 - er_t[:, b:b + 1] * ww) \
                + ag[b:b + 1, :] * wv_t[:, b:b + 1] * ww
            mem_ref[b] = upd * dec
        return jnp.concatenate(rv_rows, axis=0)

    jax.lax.fori_loop(0, T, step, jnp.zeros((B, MDIM), jnp.float32))


def _mm(a, b):
    return jax.lax.dot_general(a, b, (((1,), (0,)), ((), ())),
                               preferred_element_type=jnp.float32)


def _lnk(x, g, b):
    m = jnp.mean(x, axis=-1, keepdims=True)
    c = x - m
    v = jnp.mean(c * c, axis=-1, keepdims=True)
    return c / jnp.sqrt(v + 1e-5) * g + b


def _controller_body(tok_ref, emb_ref, pos_ref, win_ref, lng_ref, lnb_ref,
                     w4_ref, wf1_ref, bf1_ref, wf2_ref, bf2_ref, out_ref):
    dh = HDIM // NHATTN
    sub_iota = jax.lax.broadcasted_iota(jnp.int32, (8, 1), 0)
    rows = []
    for b in range(B):
        for t in range(T):
            tok = tok_ref[b, t]
            blk = emb_ref[pl.ds(pl.multiple_of((tok // 8) * 8, 8), 8), :]
            row = jnp.sum(jnp.where(sub_iota == tok % 8, blk, 0.0),
                          axis=0, keepdims=True)
            rows.append(row + pos_ref[t:t + 1, :])
    x = _mm(jnp.concatenate(rows, axis=0), win_ref[...])

    q_iota = jax.lax.broadcasted_iota(jnp.int32, (T, T), 0)
    k_iota = jax.lax.broadcasted_iota(jnp.int32, (T, T), 1)
    causal = q_iota >= k_iota

    for l in range(2):
        h = _lnk(x, lng_ref[4 * l:4 * l + 1], lnb_ref[4 * l:4 * l + 1])
        q = _mm(h, w4_ref[4 * l + 0])
        k = _mm(h, w4_ref[4 * l + 1])
        v = _mm(h, w4_ref[4 * l + 2])
        o_rows = []
        for b in range(B):
            o_lanes = []
            for hh in range(NHATTN):
                qs = q[T * b:T * b + T, dh * hh:dh * hh + dh]
                ks = k[T * b:T * b + T, dh * hh:dh * hh + dh]
                vs = v[T * b:T * b + T, dh * hh:dh * hh + dh]
                s = jax.lax.dot_general(
                    qs, ks, (((1,), (1,)), ((), ())),
                    preferred_element_type=jnp.float32) * (1.0 / float(np.sqrt(dh)))
                s = jnp.where(causal, s, jnp.float32(-1e9))
                s = s - jnp.max(s, axis=1, keepdims=True)
                e = jnp.exp(s)
                a = e / jnp.sum(e, axis=1, keepdims=True)
                o_lanes.append(_mm(a, vs))
            o_rows.append(jnp.concatenate(o_lanes, axis=1))
        o = jnp.concatenate(o_rows, axis=0)
        x = x + _mm(o, w4_ref[4 * l + 3])
        h2 = _lnk(x, lng_ref[4 * l + 2:4 * l + 3], lnb_ref[4 * l + 2:4 * l + 3])
        g = _mm(h2, wf1_ref[l]) + bf1_ref[l]
        g = jax.nn.gelu(g)
        x = x + _mm(g, wf2_ref[l]) + bf2_ref[l]
    out_ref[...] = _lnk(x, lng_ref[1:2], lnb_ref[1:2])


def _controller_hidden(params, tokens, interpret=False):
    L = params['layers']
    lng = jnp.stack([L[0]['ln1_g'], params['lnf_g'], L[0]['ln2_g'],
                     jnp.zeros((HDIM,), jnp.float32),
                     L[1]['ln1_g'], jnp.zeros((HDIM,), jnp.float32),
                     L[1]['ln2_g'], jnp.zeros((HDIM,), jnp.float32)])
    lnb = jnp.stack([L[0]['ln1_b'], params['lnf_b'], L[0]['ln2_b'],
                     jnp.zeros((HDIM,), jnp.float32),
                     L[1]['ln1_b'], jnp.zeros((HDIM,), jnp.float32),
                     L[1]['ln2_b'], jnp.zeros((HDIM,), jnp.float32)])
    w4 = jnp.stack([L[0]['Wq'], L[0]['Wk'], L[0]['Wv'], L[0]['Wo'],
                    L[1]['Wq'], L[1]['Wk'], L[1]['Wv'], L[1]['Wo']])
    wf1 = jnp.stack([L[0]['W1'], L[1]['W1']])
    bf1 = jnp.stack([L[0]['b1'].reshape(1, DFF), L[1]['b1'].reshape(1, DFF)])
    wf2 = jnp.stack([L[0]['W2'], L[1]['W2']])
    bf2 = jnp.stack([L[0]['b2'].reshape(1, HDIM), L[1]['b2'].reshape(1, HDIM)])
    grid_spec = pltpu.PrefetchScalarGridSpec(num_scalar_prefetch=1)
    x = pl.pallas_call(
        _controller_body,
        out_shape=jax.ShapeDtypeStruct((B * T, HDIM), jnp.float32),
        grid_spec=grid_spec,
        interpret=interpret,
    )(tokens, params['embed'], params['pos'], params['W_in'],
      lng, lnb, w4, wf1, bf1, wf2, bf2)
    return x.reshape(B, T, HDIM)


def _build_wsmall(params):
    beta_r = jnp.clip(jax.nn.softplus(params['beta_read']), 1.0, 20.0)
    beta_w = jnp.clip(jax.nn.softplus(params['beta_write']), 1.0, 20.0)
    C = HDIM + MDIM
    w = jnp.zeros((C, NCOL), jnp.float32)
    w = w.at[:, C_RK:C_RK + MHEADS * MDIM].set(params['W_rk'] * beta_r)
    w = w.at[:, C_WK:C_WK + MDIM].set(params['W_wk'] * beta_w)
    w = w.at[:, C_WV:C_WV + MDIM].set(params['W_wv'])
    w = w.at[:, C_ER:C_ER + MDIM].set(params['W_er'])
    w = w.at[:, C_AG:C_AG + 1].set(params['W_ag'])
    return w


@functools.partial(jax.jit, static_argnames=('interpret',))
def kernel(input_seq, params, interpret=False):
    h = _controller_hidden(params, input_seq, interpret)
    h_tm = jnp.transpose(h, (1, 0, 2)).reshape(B * T, HDIM)
    ws = _build_wsmall(params)
    bias = params['b_logits'].reshape(1, VOCAB)
    dec = jax.nn.sigmoid(params['decay']).reshape(1, 1)

    out = pl.pallas_call(
        _scan_body,
        out_shape=jax.ShapeDtypeStruct((T, B, VOCAB), jnp.float32),
        scratch_shapes=[
            pltpu.VMEM((B, MDIM, SLOTS), jnp.float32),
            pltpu.VMEM((T, B, VOCAB), jnp.float32),
            pltpu.VMEM((T, B, NCOL), jnp.float32),
        ],
        interpret=interpret,
    )(h_tm, params['W_logits'], ws, bias, dec)
    return jnp.transpose(out, (1, 0, 2))


# speedup vs baseline: 1.0902x; 1.0902x over previous
import functools

import jax
import jax.numpy as jnp
import numpy as np
from jax.experimental import pallas as pl
from jax.experimental.pallas import tpu as pltpu

SLOTS = 1024
MDIM = 64
MHEADS = 4
TOPK = 32
VOCAB = 8192
EDIM = 512
HDIM = 512
NHATTN = 8
DFF = 2048
B = 4
T = 32

C_RK = 0
C_WK = 256
C_WV = 384
C_ER = 512
C_AG = 640
NCOL = 768
KEY_OFFS = (C_RK, C_RK + 64, C_RK + 128, C_RK + 192, C_WK)


def _scan_body(h_ref, wl_ref, ws_ref, bias_ref, dec_ref, out_ref,
               mem_ref, plog_ref, psm_ref):
    h = h_ref[...]
    plog_ref[...] = (jax.lax.dot_general(
        h, wl_ref[:HDIM], (((1,), (0,)), ((), ())),
        preferred_element_type=jnp.float32) + bias_ref[...]).reshape(T, B, VOCAB)
    psm_ref[...] = jax.lax.dot_general(
        h, ws_ref[:HDIM], (((1,), (0,)), ((), ())),
        preferred_element_type=jnp.float32).reshape(T, B, NCOL)
    mem_ref[...] = jnp.zeros((B, MDIM, SLOTS), jnp.float32)
    dec = dec_ref[...]
    wl2 = wl_ref[HDIM:]
    ws2 = ws_ref[HDIM:]

    ones_col = jnp.ones((SLOTS, 1), jnp.float32)

    def step(t, rv):
        out_ref[t] = plog_ref[t] + jax.lax.dot_general(
            rv, wl2, (((1,), (0,)), ((), ())), preferred_element_type=jnp.float32)
        pr = psm_ref[t] + jax.lax.dot_general(
            rv, ws2, (((1,), (0,)), ((), ())),
            preferred_element_type=jnp.float32)

        s_rows = []
        for b in range(B):
            kb = jnp.concatenate(
                [pr[b:b + 1, o:o + MDIM] for o in KEY_OFFS], axis=0)
            s_rows.append(jax.lax.dot_general(
                kb, mem_ref[b], (((1,), (0,)), ((), ())),
                preferred_element_type=jnp.float32))
        s_orig = jnp.concatenate(s_rows, axis=0)

        bits = jax.lax.bitcast_convert_type(s_orig, jnp.int32)
        okey = jnp.where(bits < 0, bits ^ jnp.int32(0x7FFFFFFF), bits)
        SIGN = jnp.int32(-2147483648)
        theta_u = jnp.zeros((5 * B, 1), jnp.int32)
        for k in range(31, -1, -1):
            cand_u = theta_u | jnp.int32(np.int32(np.uint32(1 << k)))
            cand_o = cand_u ^ SIGN
            cnt = jax.lax.dot_general(
                jnp.where(okey >= cand_o, 1.0, 0.0), ones_col,
                (((1,), (0,)), ((), ())),
                preferred_element_type=jnp.float32)
            theta_u = jnp.where(cnt >= float(TOPK), cand_u, theta_u)
        theta_o = theta_u ^ SIGN
        gt = okey > theta_o
        eq = okey == theta_o
        n_gt = jax.lax.dot_general(
            jnp.where(gt, 1.0, 0.0), ones_col, (((1,), (0,)), ((), ())),
            preferred_element_type=jnp.float32)
        eqf = jnp.where(eq, 1.0, 0.0)
        run = eqf
        for sh in (1, 2, 4, 8, 16, 32, 64, 128, 256, 512):
            run = run + jnp.concatenate(
                [jnp.zeros((5 * B, sh), jnp.float32), run[:, :-sh]], axis=1)
        rank = run - eqf
        q = jnp.float32(TOPK) - n_gt
        sel = jnp.where(jnp.logical_or(gt, jnp.logical_and(eq, rank < q)),
                        1.0, 0.0)

        gmax = jnp.max(s_orig, axis=1, keepdims=True)
        w_un = sel * jnp.exp(s_orig - gmax)
        wf = w_un / jnp.sum(w_un, axis=1, keepdims=True)

        wv_t = jnp.transpose(pr[:, C_WV:C_WV + MDIM])
        er_t = jnp.transpose(jax.nn.sigmoid(pr[:, C_ER:C_ER + MDIM]))
        ag = jax.nn.sigmoid(pr[:, C_AG:C_AG + 1])

        rv_rows = []
        for b in range(B):
            mb = mem_ref[b]
            wr = wf[5 * b:5 * b + MHEADS]
            rb = jax.lax.dot_general(
                wr, mb, (((1,), (1,)), ((), ())),
                preferred_element_type=jnp.float32)
            rv_rows.append(jnp.mean(rb, axis=0, keepdims=True))
            ww = wf[5 * b + MHEADS:5 * b + MHEADS + 1]
            upd = mb * (1.0 - er_t[:, b:b + 1] * ww) \
                + ag[b:b + 1, :] * wv_t[:, b:b + 1] * ww
            mem_ref[b] = upd * dec
        return jnp.concatenate(rv_rows, axis=0)

    jax.lax.fori_loop(0, T, step, jnp.zeros((B, MDIM), jnp.float32))


def _mm(a, b):
    return jax.lax.dot_general(a, b, (((1,), (0,)), ((), ())),
                               preferred_element_type=jnp.float32)


def _lnk(x, g, b):
    m = jnp.mean(x, axis=-1, keepdims=True)
    c = x - m
    v = jnp.mean(c * c, axis=-1, keepdims=True)
    return c / jnp.sqrt(v + 1e-5) * g + b


def _controller_body(tok_ref, emb_ref, pos_ref, win_ref, lng_ref, lnb_ref,
                     w4_ref, wf1_ref, bf1_ref, wf2_ref, bf2_ref, out_ref):
    dh = HDIM // NHATTN
    sub_iota = jax.lax.broadcasted_iota(jnp.int32, (8, 1), 0)
    rows = []
    for b in range(B):
        for t in range(T):
            tok = tok_ref[b, t]
            blk = emb_ref[pl.ds(pl.multiple_of((tok // 8) * 8, 8), 8), :]
            row = jnp.sum(jnp.where(sub_iota == tok % 8, blk, 0.0),
                          axis=0, keepdims=True)
            rows.append(row + pos_ref[t:t + 1, :])
    x = _mm(jnp.concatenate(rows, axis=0), win_ref[...])

    q_iota = jax.lax.broadcasted_iota(jnp.int32, (T, T), 0)
    k_iota = jax.lax.broadcasted_iota(jnp.int32, (T, T), 1)
    causal = q_iota >= k_iota

    for l in range(2):
        h = _lnk(x, lng_ref[4 * l:4 * l + 1], lnb_ref[4 * l:4 * l + 1])
        q = _mm(h, w4_ref[4 * l + 0])
        k = _mm(h, w4_ref[4 * l + 1])
        v = _mm(h, w4_ref[4 * l + 2])
        o_rows = []
        for b in range(B):
            o_lanes = []
            for hh in range(NHATTN):
                qs = q[T * b:T * b + T, dh * hh:dh * hh + dh]
                ks = k[T * b:T * b + T, dh * hh:dh * hh + dh]
                vs = v[T * b:T * b + T, dh * hh:dh * hh + dh]
                s = jax.lax.dot_general(
                    qs, ks, (((1,), (1,)), ((), ())),
                    preferred_element_type=jnp.float32) * (1.0 / float(np.sqrt(dh)))
                s = jnp.where(causal, s, jnp.float32(-1e9))
                s = s - jnp.max(s, axis=1, keepdims=True)
                e = jnp.exp(s)
                a = e / jnp.sum(e, axis=1, keepdims=True)
                o_lanes.append(_mm(a, vs))
            o_rows.append(jnp.concatenate(o_lanes, axis=1))
        o = jnp.concatenate(o_rows, axis=0)
        x = x + _mm(o, w4_ref[4 * l + 3])
        h2 = _lnk(x, lng_ref[4 * l + 2:4 * l + 3], lnb_ref[4 * l + 2:4 * l + 3])
        g = _mm(h2, wf1_ref[l]) + bf1_ref[l]
        g = jax.nn.gelu(g)
        x = x + _mm(g, wf2_ref[l]) + bf2_ref[l]
    out_ref[...] = _lnk(x, lng_ref[1:2], lnb_ref[1:2])


def _controller_hidden(params, tokens, interpret=False):
    L = params['layers']
    lng = jnp.stack([L[0]['ln1_g'], params['lnf_g'], L[0]['ln2_g'],
                     jnp.zeros((HDIM,), jnp.float32),
                     L[1]['ln1_g'], jnp.zeros((HDIM,), jnp.float32),
                     L[1]['ln2_g'], jnp.zeros((HDIM,), jnp.float32)])
    lnb = jnp.stack([L[0]['ln1_b'], params['lnf_b'], L[0]['ln2_b'],
                     jnp.zeros((HDIM,), jnp.float32),
                     L[1]['ln1_b'], jnp.zeros((HDIM,), jnp.float32),
                     L[1]['ln2_b'], jnp.zeros((HDIM,), jnp.float32)])
    w4 = jnp.stack([L[0]['Wq'], L[0]['Wk'], L[0]['Wv'], L[0]['Wo'],
                    L[1]['Wq'], L[1]['Wk'], L[1]['Wv'], L[1]['Wo']])
    wf1 = jnp.stack([L[0]['W1'], L[1]['W1']])
    bf1 = jnp.stack([L[0]['b1'].reshape(1, DFF), L[1]['b1'].reshape(1, DFF)])
    wf2 = jnp.stack([L[0]['W2'], L[1]['W2']])
    bf2 = jnp.stack([L[0]['b2'].reshape(1, HDIM), L[1]['b2'].reshape(1, HDIM)])
    grid_spec = pltpu.PrefetchScalarGridSpec(num_scalar_prefetch=1)
    x = pl.pallas_call(
        _controller_body,
        out_shape=jax.ShapeDtypeStruct((B * T, HDIM), jnp.float32),
        grid_spec=grid_spec,
        interpret=interpret,
    )(tokens, params['embed'], params['pos'], params['W_in'],
      lng, lnb, w4, wf1, bf1, wf2, bf2)
    return x.reshape(B, T, HDIM)


def _build_wsmall(params):
    beta_r = jnp.clip(jax.nn.softplus(params['beta_read']), 1.0, 20.0)
    beta_w = jnp.clip(jax.nn.softplus(params['beta_write']), 1.0, 20.0)
    C = HDIM + MDIM
    w = jnp.zeros((C, NCOL), jnp.float32)
    w = w.at[:, C_RK:C_RK + MHEADS * MDIM].set(params['W_rk'] * beta_r)
    w = w.at[:, C_WK:C_WK + MDIM].set(params['W_wk'] * beta_w)
    w = w.at[:, C_WV:C_WV + MDIM].set(params['W_wv'])
    w = w.at[:, C_ER:C_ER + MDIM].set(params['W_er'])
    w = w.at[:, C_AG:C_AG + 1].set(params['W_ag'])
    return w


@functools.partial(jax.jit, static_argnames=('interpret',))
def kernel(input_seq, params, interpret=False):
    h = _controller_hidden(params, input_seq, interpret)
    h_tm = jnp.transpose(h, (1, 0, 2)).reshape(B * T, HDIM)
    ws = _build_wsmall(params)
    bias = params['b_logits'].reshape(1, VOCAB)
    dec = jax.nn.sigmoid(params['decay']).reshape(1, 1)

    out = pl.pallas_call(
        _scan_body,
        out_shape=jax.ShapeDtypeStruct((T, B, VOCAB), jnp.float32),
        scratch_shapes=[
            pltpu.VMEM((B, MDIM, SLOTS), jnp.float32),
            pltpu.VMEM((T, B, VOCAB), jnp.float32),
            pltpu.VMEM((T, B, NCOL), jnp.float32),
        ],
        interpret=interpret,
    )(h_tm, params['W_logits'], ws, bias, dec)
    return jnp.transpose(out, (1, 0, 2))


# speedup vs baseline: 1.9018x; 1.7444x over previous
import functools

import jax
import jax.numpy as jnp
import numpy as np
from jax.experimental import pallas as pl
from jax.experimental.pallas import tpu as pltpu

SLOTS = 1024
MDIM = 64
MHEADS = 4
TOPK = 32
VOCAB = 8192
EDIM = 512
HDIM = 512
NHATTN = 8
DFF = 2048
B = 4
T = 32

C_RK = 0
C_WK = 256
C_WV = 384
C_ER = 512
C_AG = 640
NCOL = 768
KEY_OFFS = (C_RK, C_RK + 64, C_RK + 128, C_RK + 192, C_WK)


def _scan_body(h_ref, wl_ref, ws_ref, bias_ref, dec_ref, out_ref,
               mem_ref, plog_ref, psm_ref):
    h = h_ref[...]
    plog_ref[...] = (jax.lax.dot_general(
        h, wl_ref[:HDIM], (((1,), (0,)), ((), ())),
        preferred_element_type=jnp.float32) + bias_ref[...]).reshape(T, B, VOCAB)
    psm_ref[...] = jax.lax.dot_general(
        h, ws_ref[:HDIM], (((1,), (0,)), ((), ())),
        preferred_element_type=jnp.float32).reshape(T, B, NCOL)
    mem_ref[...] = jnp.zeros((B, MDIM, SLOTS), jnp.float32)
    dec = dec_ref[...]
    wl2 = wl_ref[HDIM:]
    ws2 = ws_ref[HDIM:]

    ones_col = jnp.ones((SLOTS, 1), jnp.float32)

    def step(t, rv):
        out_ref[t] = plog_ref[t] + jax.lax.dot_general(
            rv, wl2, (((1,), (0,)), ((), ())), preferred_element_type=jnp.float32)
        pr = psm_ref[t] + jax.lax.dot_general(
            rv, ws2, (((1,), (0,)), ((), ())),
            preferred_element_type=jnp.float32)

        s_rows = []
        for b in range(B):
            kb = jnp.concatenate(
                [pr[b:b + 1, o:o + MDIM] for o in KEY_OFFS], axis=0)
            s_rows.append(jax.lax.dot_general(
                kb, mem_ref[b], (((1,), (0,)), ((), ())),
                preferred_element_type=jnp.float32))
        s_orig = jnp.concatenate(s_rows, axis=0)

        bits = jax.lax.bitcast_convert_type(s_orig, jnp.int32)
        okey = jnp.where(bits < 0, bits ^ jnp.int32(0x7FFFFFFF), bits)
        SIGN = jnp.int32(-2147483648)
        theta_u = jnp.zeros((5 * B, 1), jnp.int32)
        for k in range(30, -1, -2):
            cands = [theta_u | jnp.int32(np.int32(np.uint32(j << k)))
                     for j in (1, 2, 3)]
            cnts = [jnp.sum(jnp.where(okey >= (c ^ SIGN), 1.0, 0.0),
                            axis=1, keepdims=True) for c in cands]
            theta_u = jnp.where(
                cnts[2] >= float(TOPK), cands[2],
                jnp.where(cnts[1] >= float(TOPK), cands[1],
                          jnp.where(cnts[0] >= float(TOPK), cands[0], theta_u)))
        theta_o = theta_u ^ SIGN
        gt = okey > theta_o
        eq = okey == theta_o
        n_gt = jnp.sum(jnp.where(gt, 1.0, 0.0), axis=1, keepdims=True)
        eqf = jnp.where(eq, 1.0, 0.0)
        run = eqf
        for sh in (1, 2, 4, 8, 16, 32, 64, 128, 256, 512):
            run = run + jnp.concatenate(
                [jnp.zeros((5 * B, sh), jnp.float32), run[:, :-sh]], axis=1)
        rank = run - eqf
        q = jnp.float32(TOPK) - n_gt
        sel = jnp.where(jnp.logical_or(gt, jnp.logical_and(eq, rank < q)),
                        1.0, 0.0)

        gmax = jnp.max(s_orig, axis=1, keepdims=True)
        w_un = sel * jnp.exp(s_orig - gmax)
        wf = w_un / jnp.sum(w_un, axis=1, keepdims=True)

        wv_t = jnp.transpose(pr[:, C_WV:C_WV + MDIM])
        er_t = jnp.transpose(jax.nn.sigmoid(pr[:, C_ER:C_ER + MDIM]))
        ag = jax.nn.sigmoid(pr[:, C_AG:C_AG + 1])

        rv_rows = []
        for b in range(B):
            mb = mem_ref[b]
            wr = wf[5 * b:5 * b + MHEADS]
            rb = jax.lax.dot_general(
                wr, mb, (((1,), (1,)), ((), ())),
                preferred_element_type=jnp.float32)
            rv_rows.append(jnp.mean(rb, axis=0, keepdims=True))
            ww = wf[5 * b + MHEADS:5 * b + MHEADS + 1]
            upd = mb * (1.0 - er_t[:, b:b + 1] * ww) \
                + ag[b:b + 1, :] * wv_t[:, b:b + 1] * ww
            mem_ref[b] = upd * dec
        return jnp.concatenate(rv_rows, axis=0)

    jax.lax.fori_loop(0, T, step, jnp.zeros((B, MDIM), jnp.float32))


def _mm(a, b):
    return jax.lax.dot_general(a, b, (((1,), (0,)), ((), ())),
                               preferred_element_type=jnp.float32)


def _lnk(x, g, b):
    m = jnp.mean(x, axis=-1, keepdims=True)
    c = x - m
    v = jnp.mean(c * c, axis=-1, keepdims=True)
    return c / jnp.sqrt(v + 1e-5) * g + b


def _controller_body(tok_ref, emb_ref, pos_ref, win_ref, lng_ref, lnb_ref,
                     w4_ref, wf1_ref, bf1_ref, wf2_ref, bf2_ref, out_ref):
    dh = HDIM // NHATTN
    sub_iota = jax.lax.broadcasted_iota(jnp.int32, (8, 1), 0)
    rows = []
    for b in range(B):
        for t in range(T):
            tok = tok_ref[b, t]
            blk = emb_ref[pl.ds(pl.multiple_of((tok // 8) * 8, 8), 8), :]
            row = jnp.sum(jnp.where(sub_iota == tok % 8, blk, 0.0),
                          axis=0, keepdims=True)
            rows.append(row + pos_ref[t:t + 1, :])
    x = _mm(jnp.concatenate(rows, axis=0), win_ref[...])

    q_iota = jax.lax.broadcasted_iota(jnp.int32, (T, T), 0)
    k_iota = jax.lax.broadcasted_iota(jnp.int32, (T, T), 1)
    causal = q_iota >= k_iota

    for l in range(2):
        h = _lnk(x, lng_ref[4 * l:4 * l + 1], lnb_ref[4 * l:4 * l + 1])
        q = _mm(h, w4_ref[4 * l + 0])
        k = _mm(h, w4_ref[4 * l + 1])
        v = _mm(h, w4_ref[4 * l + 2])
        o_rows = []
        for b in range(B):
            o_lanes = []
            for hh in range(NHATTN):
                qs = q[T * b:T * b + T, dh * hh:dh * hh + dh]
                ks = k[T * b:T * b + T, dh * hh:dh * hh + dh]
                vs = v[T * b:T * b + T, dh * hh:dh * hh + dh]
                s = jax.lax.dot_general(
                    qs, ks, (((1,), (1,)), ((), ())),
                    preferred_element_type=jnp.float32) * (1.0 / float(np.sqrt(dh)))
                s = jnp.where(causal, s, jnp.float32(-1e9))
                s = s - jnp.max(s, axis=1, keepdims=True)
                e = jnp.exp(s)
                a = e / jnp.sum(e, axis=1, keepdims=True)
                o_lanes.append(_mm(a, vs))
            o_rows.append(jnp.concatenate(o_lanes, axis=1))
        o = jnp.concatenate(o_rows, axis=0)
        x = x + _mm(o, w4_ref[4 * l + 3])
        h2 = _lnk(x, lng_ref[4 * l + 2:4 * l + 3], lnb_ref[4 * l + 2:4 * l + 3])
        g = _mm(h2, wf1_ref[l]) + bf1_ref[l]
        g = jax.nn.gelu(g)
        x = x + _mm(g, wf2_ref[l]) + bf2_ref[l]
    out_ref[...] = _lnk(x, lng_ref[1:2], lnb_ref[1:2])


def _controller_hidden(params, tokens, interpret=False):
    L = params['layers']
    lng = jnp.stack([L[0]['ln1_g'], params['lnf_g'], L[0]['ln2_g'],
                     jnp.zeros((HDIM,), jnp.float32),
                     L[1]['ln1_g'], jnp.zeros((HDIM,), jnp.float32),
                     L[1]['ln2_g'], jnp.zeros((HDIM,), jnp.float32)])
    lnb = jnp.stack([L[0]['ln1_b'], params['lnf_b'], L[0]['ln2_b'],
                     jnp.zeros((HDIM,), jnp.float32),
                     L[1]['ln1_b'], jnp.zeros((HDIM,), jnp.float32),
                     L[1]['ln2_b'], jnp.zeros((HDIM,), jnp.float32)])
    w4 = jnp.stack([L[0]['Wq'], L[0]['Wk'], L[0]['Wv'], L[0]['Wo'],
                    L[1]['Wq'], L[1]['Wk'], L[1]['Wv'], L[1]['Wo']])
    wf1 = jnp.stack([L[0]['W1'], L[1]['W1']])
    bf1 = jnp.stack([L[0]['b1'].reshape(1, DFF), L[1]['b1'].reshape(1, DFF)])
    wf2 = jnp.stack([L[0]['W2'], L[1]['W2']])
    bf2 = jnp.stack([L[0]['b2'].reshape(1, HDIM), L[1]['b2'].reshape(1, HDIM)])
    grid_spec = pltpu.PrefetchScalarGridSpec(num_scalar_prefetch=1)
    x = pl.pallas_call(
        _controller_body,
        out_shape=jax.ShapeDtypeStruct((B * T, HDIM), jnp.float32),
        grid_spec=grid_spec,
        interpret=interpret,
    )(tokens, params['embed'], params['pos'], params['W_in'],
      lng, lnb, w4, wf1, bf1, wf2, bf2)
    return x.reshape(B, T, HDIM)


def _build_wsmall(params):
    beta_r = jnp.clip(jax.nn.softplus(params['beta_read']), 1.0, 20.0)
    beta_w = jnp.clip(jax.nn.softplus(params['beta_write']), 1.0, 20.0)
    C = HDIM + MDIM
    w = jnp.zeros((C, NCOL), jnp.float32)
    w = w.at[:, C_RK:C_RK + MHEADS * MDIM].set(params['W_rk'] * beta_r)
    w = w.at[:, C_WK:C_WK + MDIM].set(params['W_wk'] * beta_w)
    w = w.at[:, C_WV:C_WV + MDIM].set(params['W_wv'])
    w = w.at[:, C_ER:C_ER + MDIM].set(params['W_er'])
    w = w.at[:, C_AG:C_AG + 1].set(params['W_ag'])
    return w


@functools.partial(jax.jit, static_argnames=('interpret',))
def kernel(input_seq, params, interpret=False):
    h = _controller_hidden(params, input_seq, interpret)
    h_tm = jnp.transpose(h, (1, 0, 2)).reshape(B * T, HDIM)
    ws = _build_wsmall(params)
    bias = params['b_logits'].reshape(1, VOCAB)
    dec = jax.nn.sigmoid(params['decay']).reshape(1, 1)

    out = pl.pallas_call(
        _scan_body,
        out_shape=jax.ShapeDtypeStruct((T, B, VOCAB), jnp.float32),
        scratch_shapes=[
            pltpu.VMEM((B, MDIM, SLOTS), jnp.float32),
            pltpu.VMEM((T, B, VOCAB), jnp.float32),
            pltpu.VMEM((T, B, NCOL), jnp.float32),
        ],
        interpret=interpret,
    )(h_tm, params['W_logits'], ws, bias, dec)
    return jnp.transpose(out, (1, 0, 2))


# speedup vs baseline: 2.1295x; 1.1197x over previous
import functools

import jax
import jax.numpy as jnp
import numpy as np
from jax.experimental import pallas as pl
from jax.experimental.pallas import tpu as pltpu

SLOTS = 1024
MDIM = 64
MHEADS = 4
TOPK = 32
VOCAB = 8192
EDIM = 512
HDIM = 512
NHATTN = 8
DFF = 2048
B = 4
T = 32

C_RK = 0
C_WK = 256
C_WV = 384
C_ER = 512
C_AG = 640
NCOL = 768
KEY_OFFS = (C_RK, C_RK + 64, C_RK + 128, C_RK + 192, C_WK)


def _scan_body(h_ref, wl_ref, wrk_ref, wwk_ref, wwv_ref, wer_ref, wag_ref,
               bias_ref, br_ref, bw_ref, dec_ref, out_ref,
               mem_ref, plog_ref, psm_ref, ws_ref):
    br = jnp.clip(jnp.log(1.0 + jnp.exp(br_ref[...])), 1.0, 20.0)
    bw = jnp.clip(jnp.log(1.0 + jnp.exp(bw_ref[...])), 1.0, 20.0)
    dec = 1.0 / (1.0 + jnp.exp(-dec_ref[...]))
    ws_ref[...] = jnp.zeros((HDIM + MDIM, NCOL), jnp.float32)
    ws_ref[:, C_RK:C_RK + MHEADS * MDIM] = wrk_ref[...] * br
    ws_ref[:, C_WK:C_WK + MDIM] = wwk_ref[...] * bw
    ws_ref[:, C_WV:C_WV + MDIM] = wwv_ref[...]
    ws_ref[:, C_ER:C_ER + MDIM] = wer_ref[...]
    ws_ref[:, C_AG:C_AG + 1] = wag_ref[...]
    h = h_ref[...]
    plog_ref[...] = (jax.lax.dot_general(
        h, wl_ref[:HDIM], (((1,), (0,)), ((), ())),
        preferred_element_type=jnp.float32) + bias_ref[...]).reshape(T, B, VOCAB)
    psm_ref[...] = jax.lax.dot_general(
        h, ws_ref[:HDIM], (((1,), (0,)), ((), ())),
        preferred_element_type=jnp.float32).reshape(T, B, NCOL)
    mem_ref[...] = jnp.zeros((B, MDIM, SLOTS), jnp.float32)
    wl2 = wl_ref[HDIM:]
    ws2 = ws_ref[HDIM:]

    def step(t, rv):
        lg = plog_ref[t] + jax.lax.dot_general(
            rv, wl2, (((1,), (0,)), ((), ())), preferred_element_type=jnp.float32)
        for b in range(B):
            out_ref[b, pl.ds(t, 1)] = lg[b:b + 1]
        pr = psm_ref[t] + jax.lax.dot_general(
            rv, ws2, (((1,), (0,)), ((), ())),
            preferred_element_type=jnp.float32)

        s_rows = []
        for b in range(B):
            kb = jnp.concatenate(
                [pr[b:b + 1, o:o + MDIM] for o in KEY_OFFS], axis=0)
            s_rows.append(jax.lax.dot_general(
                kb, mem_ref[b], (((1,), (0,)), ((), ())),
                preferred_element_type=jnp.float32))
        s_orig = jnp.concatenate(s_rows, axis=0)

        bits = jax.lax.bitcast_convert_type(s_orig, jnp.int32)
        okey = jnp.where(bits < 0, bits ^ jnp.int32(0x7FFFFFFF), bits)
        SIGN = jnp.int32(-2147483648)
        theta_u = jnp.zeros((5 * B, 1), jnp.int32)
        for k in range(30, -1, -2):
            cands = [theta_u | jnp.int32(np.int32(np.uint32(j << k)))
                     for j in (1, 2, 3)]
            cnts = [jnp.sum(jnp.where(okey >= (c ^ SIGN), 1.0, 0.0),
                            axis=1, keepdims=True) for c in cands]
            theta_u = jnp.where(
                cnts[2] >= float(TOPK), cands[2],
                jnp.where(cnts[1] >= float(TOPK), cands[1],
                          jnp.where(cnts[0] >= float(TOPK), cands[0], theta_u)))
        theta_o = theta_u ^ SIGN
        gt = okey > theta_o
        eq = okey == theta_o
        n_gt = jnp.sum(jnp.where(gt, 1.0, 0.0), axis=1, keepdims=True)
        eqf = jnp.where(eq, 1.0, 0.0)
        run = eqf
        for sh in (1, 2, 4, 8, 16, 32, 64, 128, 256, 512):
            run = run + jnp.concatenate(
                [jnp.zeros((5 * B, sh), jnp.float32), run[:, :-sh]], axis=1)
        rank = run - eqf
        q = jnp.float32(TOPK) - n_gt
        sel = jnp.where(jnp.logical_or(gt, jnp.logical_and(eq, rank < q)),
                        1.0, 0.0)

        gmax = jnp.max(s_orig, axis=1, keepdims=True)
        w_un = sel * jnp.exp(s_orig - gmax)
        wf = w_un / jnp.sum(w_un, axis=1, keepdims=True)

        wv_t = jnp.transpose(pr[:, C_WV:C_WV + MDIM])
        er_t = jnp.transpose(jax.nn.sigmoid(pr[:, C_ER:C_ER + MDIM]))
        ag = jax.nn.sigmoid(pr[:, C_AG:C_AG + 1])

        rv_rows = []
        for b in range(B):
            mb = mem_ref[b]
            wr = wf[5 * b:5 * b + MHEADS]
            rb = jax.lax.dot_general(
                wr, mb, (((1,), (1,)), ((), ())),
                preferred_element_type=jnp.float32)
            rv_rows.append(jnp.mean(rb, axis=0, keepdims=True))
            ww = wf[5 * b + MHEADS:5 * b + MHEADS + 1]
            upd = mb * (1.0 - er_t[:, b:b + 1] * ww) \
                + ag[b:b + 1, :] * wv_t[:, b:b + 1] * ww
            mem_ref[b] = upd * dec
        return jnp.concatenate(rv_rows, axis=0)

    jax.lax.fori_loop(0, T, step, jnp.zeros((B, MDIM), jnp.float32))


def _mm(a, b):
    return jax.lax.dot_general(a, b, (((1,), (0,)), ((), ())),
                               preferred_element_type=jnp.float32)


def _lnk(x, g, b):
    m = jnp.mean(x, axis=-1, keepdims=True)
    c = x - m
    v = jnp.mean(c * c, axis=-1, keepdims=True)
    return c / jnp.sqrt(v + 1e-5) * g + b


def _controller_body(tok_ref, emb_ref, pos_ref, win_ref, lng_ref, lnb_ref,
                     w4_ref, wf1_ref, bf1_ref, wf2_ref, bf2_ref, out_ref):
    dh = HDIM // NHATTN
    sub_iota = jax.lax.broadcasted_iota(jnp.int32, (8, 1), 0)
    rows = []
    for b in range(B):
        for t in range(T):
            tok = tok_ref[b, t]
            blk = emb_ref[pl.ds(pl.multiple_of((tok // 8) * 8, 8), 8), :]
            row = jnp.sum(jnp.where(sub_iota == tok % 8, blk, 0.0),
                          axis=0, keepdims=True)
            rows.append(row + pos_ref[t:t + 1, :])
    x = _mm(jnp.concatenate(rows, axis=0), win_ref[...])

    q_iota = jax.lax.broadcasted_iota(jnp.int32, (T, T), 0)
    k_iota = jax.lax.broadcasted_iota(jnp.int32, (T, T), 1)
    causal = q_iota >= k_iota

    for l in range(2):
        h = _lnk(x, lng_ref[4 * l:4 * l + 1], lnb_ref[4 * l:4 * l + 1])
        q = _mm(h, w4_ref[4 * l + 0])
        k = _mm(h, w4_ref[4 * l + 1])
        v = _mm(h, w4_ref[4 * l + 2])
        o_rows = []
        for b in range(B):
            o_lanes = []
            for hh in range(NHATTN):
                qs = q[T * b:T * b + T, dh * hh:dh * hh + dh]
                ks = k[T * b:T * b + T, dh * hh:dh * hh + dh]
                vs = v[T * b:T * b + T, dh * hh:dh * hh + dh]
                s = jax.lax.dot_general(
                    qs, ks, (((1,), (1,)), ((), ())),
                    preferred_element_type=jnp.float32) * (1.0 / float(np.sqrt(dh)))
                s = jnp.where(causal, s, jnp.float32(-1e9))
                s = s - jnp.max(s, axis=1, keepdims=True)
                e = jnp.exp(s)
                a = e / jnp.sum(e, axis=1, keepdims=True)
                o_lanes.append(_mm(a, vs))
            o_rows.append(jnp.concatenate(o_lanes, axis=1))
        o = jnp.concatenate(o_rows, axis=0)
        x = x + _mm(o, w4_ref[4 * l + 3])
        h2 = _lnk(x, lng_ref[4 * l + 2:4 * l + 3], lnb_ref[4 * l + 2:4 * l + 3])
        g = _mm(h2, wf1_ref[l]) + bf1_ref[l]
        g = jax.nn.gelu(g)
        x = x + _mm(g, wf2_ref[l]) + bf2_ref[l]
    out_ref[...] = _lnk(x, lng_ref[1:2], lnb_ref[1:2])


def _controller_hidden(params, tokens, interpret=False):
    L = params['layers']
    lng = jnp.stack([L[0]['ln1_g'], params['lnf_g'], L[0]['ln2_g'],
                     jnp.zeros((HDIM,), jnp.float32),
                     L[1]['ln1_g'], jnp.zeros((HDIM,), jnp.float32),
                     L[1]['ln2_g'], jnp.zeros((HDIM,), jnp.float32)])
    lnb = jnp.stack([L[0]['ln1_b'], params['lnf_b'], L[0]['ln2_b'],
                     jnp.zeros((HDIM,), jnp.float32),
                     L[1]['ln1_b'], jnp.zeros((HDIM,), jnp.float32),
                     L[1]['ln2_b'], jnp.zeros((HDIM,), jnp.float32)])
    w4 = jnp.stack([L[0]['Wq'], L[0]['Wk'], L[0]['Wv'], L[0]['Wo'],
                    L[1]['Wq'], L[1]['Wk'], L[1]['Wv'], L[1]['Wo']])
    wf1 = jnp.stack([L[0]['W1'], L[1]['W1']])
    bf1 = jnp.stack([L[0]['b1'].reshape(1, DFF), L[1]['b1'].reshape(1, DFF)])
    wf2 = jnp.stack([L[0]['W2'], L[1]['W2']])
    bf2 = jnp.stack([L[0]['b2'].reshape(1, HDIM), L[1]['b2'].reshape(1, HDIM)])
    grid_spec = pltpu.PrefetchScalarGridSpec(num_scalar_prefetch=1)
    x = pl.pallas_call(
        _controller_body,
        out_shape=jax.ShapeDtypeStruct((B * T, HDIM), jnp.float32),
        grid_spec=grid_spec,
        interpret=interpret,
    )(tokens, params['embed'], params['pos'], params['W_in'],
      lng, lnb, w4, wf1, bf1, wf2, bf2)
    return x.reshape(B, T, HDIM)


@functools.partial(jax.jit, static_argnames=('interpret',))
def kernel(input_seq, params, interpret=False):
    h = _controller_hidden(params, input_seq, interpret)
    h_tm = jnp.transpose(h, (1, 0, 2)).reshape(B * T, HDIM)

    return pl.pallas_call(
        _scan_body,
        out_shape=jax.ShapeDtypeStruct((B, T, VOCAB), jnp.float32),
        scratch_shapes=[
            pltpu.VMEM((B, MDIM, SLOTS), jnp.float32),
            pltpu.VMEM((T, B, VOCAB), jnp.float32),
            pltpu.VMEM((T, B, NCOL), jnp.float32),
            pltpu.VMEM((HDIM + MDIM, NCOL), jnp.float32),
        ],
        interpret=interpret,
    )(h_tm, params['W_logits'], params['W_rk'], params['W_wk'],
      params['W_wv'], params['W_er'], params['W_ag'],
      params['b_logits'].reshape(1, VOCAB),
      params['beta_read'].reshape(1, 1), params['beta_write'].reshape(1, 1),
      params['decay'].reshape(1, 1))


# speedup vs baseline: 2.2433x; 1.0534x over previous
import functools

import jax
import jax.numpy as jnp
import numpy as np
from jax.experimental import pallas as pl
from jax.experimental.pallas import tpu as pltpu

SLOTS = 1024
MDIM = 64
MHEADS = 4
TOPK = 32
VOCAB = 8192
EDIM = 512
HDIM = 512
NHATTN = 8
DFF = 2048
B = 4
T = 32

C_RK = 0
C_WK = 256
C_WV = 384
C_ER = 512
C_AG = 640
NCOL = 768
KEY_OFFS = (C_RK, C_RK + 64, C_RK + 128, C_RK + 192, C_WK)


def _scan_body(h_ref, wl_ref, wrk_ref, wwk_ref, wwv_ref, wer_ref, wag_ref,
               bias_ref, br_ref, bw_ref, dec_ref, out_ref,
               mem_ref, plog_ref, psm_ref, ws_ref):
    br = jnp.clip(jnp.log(1.0 + jnp.exp(br_ref[...])), 1.0, 20.0)
    bw = jnp.clip(jnp.log(1.0 + jnp.exp(bw_ref[...])), 1.0, 20.0)
    dec = 1.0 / (1.0 + jnp.exp(-dec_ref[...]))
    ws_ref[...] = jnp.zeros((HDIM + MDIM, NCOL), jnp.float32)
    ws_ref[:, C_RK:C_RK + MHEADS * MDIM] = wrk_ref[...] * br
    ws_ref[:, C_WK:C_WK + MDIM] = wwk_ref[...] * bw
    ws_ref[:, C_WV:C_WV + MDIM] = wwv_ref[...]
    ws_ref[:, C_ER:C_ER + MDIM] = wer_ref[...]
    ws_ref[:, C_AG:C_AG + 1] = wag_ref[...]
    h = h_ref[...]
    plog_ref[...] = (jax.lax.dot_general(
        h, wl_ref[:HDIM], (((1,), (0,)), ((), ())),
        preferred_element_type=jnp.float32) + bias_ref[...]).reshape(T, B, VOCAB)
    psm_ref[...] = jax.lax.dot_general(
        h, ws_ref[:HDIM], (((1,), (0,)), ((), ())),
        preferred_element_type=jnp.float32).reshape(T, B, NCOL)
    mem_ref[...] = jnp.zeros((B, MDIM, SLOTS), jnp.float32)
    wl2 = wl_ref[HDIM:]
    ws2 = ws_ref[HDIM:]

    def make_step(W):
        return lambda t, rv: step(t, rv, W)

    def step(t, rv, W):
        lg = plog_ref[t] + jax.lax.dot_general(
            rv, wl2, (((1,), (0,)), ((), ())), preferred_element_type=jnp.float32)
        for b in range(B):
            out_ref[b, pl.ds(t, 1)] = lg[b:b + 1]
        pr = psm_ref[t] + jax.lax.dot_general(
            rv, ws2, (((1,), (0,)), ((), ())),
            preferred_element_type=jnp.float32)

        s_rows = []
        for b in range(B):
            kb = jnp.concatenate(
                [pr[b:b + 1, o:o + MDIM] for o in KEY_OFFS], axis=0)
            s_rows.append(jax.lax.dot_general(
                kb, mem_ref[b, :, 0:W], (((1,), (0,)), ((), ())),
                preferred_element_type=jnp.float32))
        s_orig = jnp.concatenate(s_rows, axis=0)

        bits = jax.lax.bitcast_convert_type(s_orig, jnp.int32)
        okey = jnp.where(bits < 0, bits ^ jnp.int32(0x7FFFFFFF), bits)
        SIGN = jnp.int32(-2147483648)
        theta_u = jnp.zeros((5 * B, 1), jnp.int32)
        for k in range(30, -1, -2):
            cands = [theta_u | jnp.int32(np.int32(np.uint32(j << k)))
                     for j in (1, 2, 3)]
            cnts = [jnp.sum(jnp.where(okey >= (c ^ SIGN), 1.0, 0.0),
                            axis=1, keepdims=True) for c in cands]
            theta_u = jnp.where(
                cnts[2] >= float(TOPK), cands[2],
                jnp.where(cnts[1] >= float(TOPK), cands[1],
                          jnp.where(cnts[0] >= float(TOPK), cands[0], theta_u)))
        theta_o = theta_u ^ SIGN
        gt = okey > theta_o
        eq = okey == theta_o
        n_gt = jnp.sum(jnp.where(gt, 1.0, 0.0), axis=1, keepdims=True)
        eqf = jnp.where(eq, 1.0, 0.0)
        run = eqf
        sh = 1
        while sh < W:
            run = run + jnp.concatenate(
                [jnp.zeros((5 * B, sh), jnp.float32), run[:, :-sh]], axis=1)
            sh *= 2
        rank = run - eqf
        q = jnp.float32(TOPK) - n_gt
        sel = jnp.where(jnp.logical_or(gt, jnp.logical_and(eq, rank < q)),
                        1.0, 0.0)

        gmax = jnp.max(s_orig, axis=1, keepdims=True)
        w_un = sel * jnp.exp(s_orig - gmax)
        wf = w_un / jnp.sum(w_un, axis=1, keepdims=True)

        wv_t = jnp.transpose(pr[:, C_WV:C_WV + MDIM])
        er_t = jnp.transpose(jax.nn.sigmoid(pr[:, C_ER:C_ER + MDIM]))
        ag = jax.nn.sigmoid(pr[:, C_AG:C_AG + 1])

        rv_rows = []
        for b in range(B):
            mb = mem_ref[b, :, 0:W]
            wr = wf[5 * b:5 * b + MHEADS]
            rb = jax.lax.dot_general(
                wr, mb, (((1,), (1,)), ((), ())),
                preferred_element_type=jnp.float32)
            rv_rows.append(jnp.mean(rb, axis=0, keepdims=True))
            ww = wf[5 * b + MHEADS:5 * b + MHEADS + 1]
            upd = mb * (1.0 - er_t[:, b:b + 1] * ww) \
                + ag[b:b + 1, :] * wv_t[:, b:b + 1] * ww
            mem_ref[b, :, 0:W] = upd * dec
        return jnp.concatenate(rv_rows, axis=0)

    rv = jnp.zeros((B, MDIM), jnp.float32)
    for p in range(4):
        rv = jax.lax.fori_loop(8 * p, 8 * p + 8, make_step(256 * (p + 1)), rv)


def _mm(a, b):
    return jax.lax.dot_general(a, b, (((1,), (0,)), ((), ())),
                               preferred_element_type=jnp.float32)


def _lnk(x, g, b):
    m = jnp.mean(x, axis=-1, keepdims=True)
    c = x - m
    v = jnp.mean(c * c, axis=-1, keepdims=True)
    return c / jnp.sqrt(v + 1e-5) * g + b


def _controller_body(tok_ref, emb_ref, pos_ref, win_ref, lng_ref, lnb_ref,
                     w4_ref, wf1_ref, bf1_ref, wf2_ref, bf2_ref, out_ref):
    dh = HDIM // NHATTN
    sub_iota = jax.lax.broadcasted_iota(jnp.int32, (8, 1), 0)
    rows = []
    for b in range(B):
        for t in range(T):
            tok = tok_ref[b, t]
            blk = emb_ref[pl.ds(pl.multiple_of((tok // 8) * 8, 8), 8), :]
            row = jnp.sum(jnp.where(sub_iota == tok % 8, blk, 0.0),
                          axis=0, keepdims=True)
            rows.append(row + pos_ref[t:t + 1, :])
    x = _mm(jnp.concatenate(rows, axis=0), win_ref[...])

    q_iota = jax.lax.broadcasted_iota(jnp.int32, (T, T), 0)
    k_iota = jax.lax.broadcasted_iota(jnp.int32, (T, T), 1)
    causal = q_iota >= k_iota

    for l in range(2):
        h = _lnk(x, lng_ref[4 * l:4 * l + 1], lnb_ref[4 * l:4 * l + 1])
        q = _mm(h, w4_ref[4 * l + 0])
        k = _mm(h, w4_ref[4 * l + 1])
        v = _mm(h, w4_ref[4 * l + 2])
        o_rows = []
        for b in range(B):
            o_lanes = []
            for hh in range(NHATTN):
                qs = q[T * b:T * b + T, dh * hh:dh * hh + dh]
                ks = k[T * b:T * b + T, dh * hh:dh * hh + dh]
                vs = v[T * b:T * b + T, dh * hh:dh * hh + dh]
                s = jax.lax.dot_general(
                    qs, ks, (((1,), (1,)), ((), ())),
                    preferred_element_type=jnp.float32) * (1.0 / float(np.sqrt(dh)))
                s = jnp.where(causal, s, jnp.float32(-1e9))
                s = s - jnp.max(s, axis=1, keepdims=True)
                e = jnp.exp(s)
                a = e / jnp.sum(e, axis=1, keepdims=True)
                o_lanes.append(_mm(a, vs))
            o_rows.append(jnp.concatenate(o_lanes, axis=1))
        o = jnp.concatenate(o_rows, axis=0)
        x = x + _mm(o, w4_ref[4 * l + 3])
        h2 = _lnk(x, lng_ref[4 * l + 2:4 * l + 3], lnb_ref[4 * l + 2:4 * l + 3])
        g = _mm(h2, wf1_ref[l]) + bf1_ref[l]
        g = jax.nn.gelu(g)
        x = x + _mm(g, wf2_ref[l]) + bf2_ref[l]
    out_ref[...] = _lnk(x, lng_ref[1:2], lnb_ref[1:2])


def _controller_hidden(params, tokens, interpret=False):
    L = params['layers']
    lng = jnp.stack([L[0]['ln1_g'], params['lnf_g'], L[0]['ln2_g'],
                     jnp.zeros((HDIM,), jnp.float32),
                     L[1]['ln1_g'], jnp.zeros((HDIM,), jnp.float32),
                     L[1]['ln2_g'], jnp.zeros((HDIM,), jnp.float32)])
    lnb = jnp.stack([L[0]['ln1_b'], params['lnf_b'], L[0]['ln2_b'],
                     jnp.zeros((HDIM,), jnp.float32),
                     L[1]['ln1_b'], jnp.zeros((HDIM,), jnp.float32),
                     L[1]['ln2_b'], jnp.zeros((HDIM,), jnp.float32)])
    w4 = jnp.stack([L[0]['Wq'], L[0]['Wk'], L[0]['Wv'], L[0]['Wo'],
                    L[1]['Wq'], L[1]['Wk'], L[1]['Wv'], L[1]['Wo']])
    wf1 = jnp.stack([L[0]['W1'], L[1]['W1']])
    bf1 = jnp.stack([L[0]['b1'].reshape(1, DFF), L[1]['b1'].reshape(1, DFF)])
    wf2 = jnp.stack([L[0]['W2'], L[1]['W2']])
    bf2 = jnp.stack([L[0]['b2'].reshape(1, HDIM), L[1]['b2'].reshape(1, HDIM)])
    grid_spec = pltpu.PrefetchScalarGridSpec(num_scalar_prefetch=1)
    x = pl.pallas_call(
        _controller_body,
        out_shape=jax.ShapeDtypeStruct((B * T, HDIM), jnp.float32),
        grid_spec=grid_spec,
        interpret=interpret,
    )(tokens, params['embed'], params['pos'], params['W_in'],
      lng, lnb, w4, wf1, bf1, wf2, bf2)
    return x.reshape(B, T, HDIM)


@functools.partial(jax.jit, static_argnames=('interpret',))
def kernel(input_seq, params, interpret=False):
    h = _controller_hidden(params, input_seq, interpret)
    h_tm = jnp.transpose(h, (1, 0, 2)).reshape(B * T, HDIM)

    return pl.pallas_call(
        _scan_body,
        out_shape=jax.ShapeDtypeStruct((B, T, VOCAB), jnp.float32),
        scratch_shapes=[
            pltpu.VMEM((B, MDIM, SLOTS), jnp.float32),
            pltpu.VMEM((T, B, VOCAB), jnp.float32),
            pltpu.VMEM((T, B, NCOL), jnp.float32),
            pltpu.VMEM((HDIM + MDIM, NCOL), jnp.float32),
        ],
        interpret=interpret,
    )(h_tm, params['W_logits'], params['W_rk'], params['W_wk'],
      params['W_wv'], params['W_er'], params['W_ag'],
      params['b_logits'].reshape(1, VOCAB),
      params['beta_read'].reshape(1, 1), params['beta_write'].reshape(1, 1),
      params['decay'].reshape(1, 1))
